# Initial kernel scaffold; baseline (speedup 1.0000x reference)
import functools

import jax
import jax.numpy as jnp
from jax import lax
from jax.experimental import pallas as pl
from jax.experimental.pallas import tpu as pltpu
from jax.experimental.pallas import tpu_sc as plsc

_EPS = 1e-5
_G = 64


def _enc_body(x_ref, w_ref, b_ref, o_ref):
    o_ref[...] = (
        jnp.dot(x_ref[...], w_ref[...], preferred_element_type=jnp.float32)
        + b_ref[...]
    )


def _mlp_body(relu_out, h_ref, p_ref, w1_ref, b1_ref, g1_ref, be1_ref,
              w2_ref, b2_ref, g2_ref, be2_ref, o_ref):
    p = p_ref[...]
    z = h_ref[...] + p[0] + p[1]
    u = jnp.dot(z, w1_ref[...], preferred_element_type=jnp.float32) + b1_ref[...]
    m1 = jnp.mean(u, axis=0, keepdims=True)
    v1 = jnp.mean((u - m1) ** 2, axis=0, keepdims=True)
    u = (u - m1) / jnp.sqrt(v1 + _EPS) * g1_ref[...] + be1_ref[...]
    u = jnp.maximum(u, 0.0)
    y = jnp.dot(u, w2_ref[...], preferred_element_type=jnp.float32) + b2_ref[...]
    m2 = jnp.mean(y, axis=0, keepdims=True)
    v2 = jnp.mean((y - m2) ** 2, axis=0, keepdims=True)
    y = (y - m2) / jnp.sqrt(v2 + _EPS) * g2_ref[...] + be2_ref[...]
    if relu_out:
        y = jnp.maximum(y, 0.0)
    o_ref[...] = y


def _pool_body(h_ref, b_ref, wp_ref, bp_ref, o_ref):
    h = h_ref[...]
    batch_row = b_ref[...]
    g = o_ref.shape[0]
    n = h.shape[0]
    onehot_t = (lax.broadcasted_iota(jnp.int32, (g, n), 0) == batch_row
                ).astype(jnp.float32)
    sums = jnp.dot(onehot_t, h, preferred_element_type=jnp.float32)
    counts = jnp.sum(onehot_t, axis=1, keepdims=True)
    hg = sums / jnp.maximum(counts, 1.0)
    o_ref[...] = (
        jnp.dot(hg, wp_ref[...], preferred_element_type=jnp.float32)
        + bp_ref[...]
    )


@functools.lru_cache(maxsize=None)
def _make_edge_kernel(n, d, e):
    nc, ns = 2, 16
    nw = nc * ns
    epw = e // nw
    k = 80
    nblk = epw // k
    rows_per_tile = n // ns
    zr = 125
    assert epw % k == 0 and rows_per_tile % zr == 0 and n % ns == 0

    mesh = plsc.VectorSubcoreMesh(core_axis_name="c", subcore_axis_name="s")

    def body(h_hbm, ea_hbm, src_hbm, dst_hbm, out_hbm, si, di, gth, ea, zbuf,
             acc):
        c = lax.axis_index("c")
        s = lax.axis_index("s")

        @pl.loop(0, zr)
        def _(i):
            for j in range(0, d, 16):
                zbuf[i, pl.ds(j, 16)] = jnp.zeros((16,), jnp.float32)

        row0 = s * rows_per_tile

        @pl.loop(0, rows_per_tile // zr)
        def _(r):
            pltpu.sync_copy(zbuf, acc.at[pl.ds(row0 + r * zr, zr)])

        plsc.subcore_barrier()

        base = (c * ns + s) * epw

        @pl.loop(0, nblk)
        def _(b):
            eoff = pl.multiple_of(base + b * k, 8)
            pltpu.sync_copy(src_hbm.at[pl.ds(eoff, k)], si)
            pltpu.sync_copy(dst_hbm.at[pl.ds(eoff, k)], di)
            pltpu.sync_copy(h_hbm.at[si], gth)
            pltpu.sync_copy(ea_hbm.at[pl.ds(eoff, k)], ea)

            @pl.loop(0, k)
            def _(i):
                for j in range(0, d, 16):
                    v = gth[i, pl.ds(j, 16)] + ea[i, pl.ds(j, 16)]
                    gth[i, pl.ds(j, 16)] = jnp.maximum(v, 0.0)

            pltpu.sync_copy(gth, acc.at[di], add=True)

        plsc.subcore_barrier()

        @pl.loop(0, rows_per_tile // zr)
        def _(r):
            rr = row0 + r * zr
            pltpu.sync_copy(acc.at[pl.ds(rr, zr)], out_hbm.at[c, pl.ds(rr, zr)])

    return pl.kernel(
        body,
        out_type=jax.ShapeDtypeStruct((nc, n, d), jnp.float32),
        mesh=mesh,
        scratch_types=[
            pltpu.VMEM((k,), jnp.int32),
            pltpu.VMEM((k,), jnp.int32),
            pltpu.VMEM((k, d), jnp.float32),
            pltpu.VMEM((k, d), jnp.float32),
            pltpu.VMEM((zr, d), jnp.float32),
            pltpu.VMEM_SHARED((n, d), jnp.float32),
        ],
    )


def kernel(x, edge_attr, W_enc, b_enc, W1, b1, g1, be1, W2, b2, g2, be2,
           Wp, bp, edge_index, batch):
    n, d = x.shape
    e = edge_attr.shape[0]
    num_layers = W1.shape[0]
    c_out = Wp.shape[1]

    src = edge_index[0]
    dst = edge_index[1]

    h = pl.pallas_call(
        _enc_body,
        out_shape=jax.ShapeDtypeStruct((n, d), jnp.float32),
    )(x, W_enc, b_enc.reshape(1, d))

    edge_fn = _make_edge_kernel(n, d, e)

    for l in range(num_layers):
        parts = edge_fn(h, edge_attr, src, dst)
        h = pl.pallas_call(
            functools.partial(_mlp_body, l < num_layers - 1),
            out_shape=jax.ShapeDtypeStruct((n, d), jnp.float32),
        )(h, parts, W1[l], b1[l].reshape(1, -1), g1[l].reshape(1, -1),
          be1[l].reshape(1, -1), W2[l], b2[l].reshape(1, -1),
          g2[l].reshape(1, -1), be2[l].reshape(1, -1))

    out = pl.pallas_call(
        _pool_body,
        out_shape=jax.ShapeDtypeStruct((_G, c_out), jnp.float32),
    )(h, batch.reshape(1, n), Wp, bp.reshape(1, c_out))
    return out


# speedup vs baseline: 3.0624x; 3.0624x over previous
import functools

import jax
import jax.numpy as jnp
from jax import lax
from jax.experimental import pallas as pl
from jax.experimental.pallas import tpu as pltpu
from jax.experimental.pallas import tpu_sc as plsc

_EPS = 1e-5
_G = 64


def _enc_body(x_ref, w_ref, b_ref, o_ref):
    o_ref[...] = (
        jnp.dot(x_ref[...], w_ref[...], preferred_element_type=jnp.float32)
        + b_ref[...]
    )


def _mlp_body(relu_out, h_ref, p_ref, w1_ref, b1_ref, g1_ref, be1_ref,
              w2_ref, b2_ref, g2_ref, be2_ref, o_ref):
    p = p_ref[...]
    z = h_ref[...] + p[0] + p[1]
    u = jnp.dot(z, w1_ref[...], preferred_element_type=jnp.float32) + b1_ref[...]
    m1 = jnp.mean(u, axis=0, keepdims=True)
    v1 = jnp.mean((u - m1) ** 2, axis=0, keepdims=True)
    u = (u - m1) / jnp.sqrt(v1 + _EPS) * g1_ref[...] + be1_ref[...]
    u = jnp.maximum(u, 0.0)
    y = jnp.dot(u, w2_ref[...], preferred_element_type=jnp.float32) + b2_ref[...]
    m2 = jnp.mean(y, axis=0, keepdims=True)
    v2 = jnp.mean((y - m2) ** 2, axis=0, keepdims=True)
    y = (y - m2) / jnp.sqrt(v2 + _EPS) * g2_ref[...] + be2_ref[...]
    if relu_out:
        y = jnp.maximum(y, 0.0)
    o_ref[...] = y


def _pool_body(h_ref, b_ref, wp_ref, bp_ref, o_ref):
    h = h_ref[...]
    batch_row = b_ref[...]
    g = o_ref.shape[0]
    n = h.shape[0]
    onehot_t = (lax.broadcasted_iota(jnp.int32, (g, n), 0) == batch_row
                ).astype(jnp.float32)
    sums = jnp.dot(onehot_t, h, preferred_element_type=jnp.float32)
    counts = jnp.sum(onehot_t, axis=1, keepdims=True)
    hg = sums / jnp.maximum(counts, 1.0)
    o_ref[...] = (
        jnp.dot(hg, wp_ref[...], preferred_element_type=jnp.float32)
        + bp_ref[...]
    )


@functools.lru_cache(maxsize=None)
def _make_edge_kernel(n, d, e):
    nc, ns = 2, 16
    nw = nc * ns
    epw = e // nw
    k = 80
    nblk = epw // k
    zr = 80
    nchunk = n // zr
    chunk_iters = (nchunk + ns - 1) // ns
    assert epw % k == 0 and n % zr == 0

    mesh = plsc.VectorSubcoreMesh(core_axis_name="c", subcore_axis_name="s",
                                  num_cores=nc, num_subcores=ns)

    def body(h_hbm, ea_hbm, src_hbm, dst_hbm, out_hbm, si, di, gth, ea, zbuf,
             acc):
        c = lax.axis_index("c")
        s = lax.axis_index("s")

        @pl.loop(0, zr)
        def _(i):
            for j in range(0, d, 16):
                zbuf[i, pl.ds(j, 16)] = jnp.zeros((16,), jnp.float32)

        @pl.loop(0, chunk_iters)
        def _(r):
            m = s + r * ns

            @pl.when(m < nchunk)
            def _():
                pltpu.sync_copy(zbuf, acc.at[pl.ds(m * zr, zr)])

        plsc.subcore_barrier()

        base = (c * ns + s) * epw

        @pl.loop(0, nblk)
        def _(b):
            eoff = pl.multiple_of(base + b * k, 8)
            pltpu.sync_copy(src_hbm.at[pl.ds(eoff, k)], si)
            pltpu.sync_copy(dst_hbm.at[pl.ds(eoff, k)], di)
            pltpu.sync_copy(h_hbm.at[si], gth)
            pltpu.sync_copy(ea_hbm.at[pl.ds(eoff, k)], ea)

            @pl.loop(0, k)
            def _(i):
                for j in range(0, d, 16):
                    v = gth[i, pl.ds(j, 16)] + ea[i, pl.ds(j, 16)]
                    gth[i, pl.ds(j, 16)] = jnp.maximum(v, 0.0)

            pltpu.sync_copy(gth, acc.at[di], add=True)

        plsc.subcore_barrier()

        @pl.loop(0, chunk_iters)
        def _(r):
            m = s + r * ns

            @pl.when(m < nchunk)
            def _():
                rr = pl.multiple_of(m * zr, 8)
                pltpu.sync_copy(acc.at[pl.ds(rr, zr)],
                                out_hbm.at[c, pl.ds(rr, zr)])

    return pl.kernel(
        body,
        out_type=jax.ShapeDtypeStruct((nc, n, d), jnp.float32),
        mesh=mesh,
        scratch_types=[
            pltpu.VMEM((k,), jnp.int32),
            pltpu.VMEM((k,), jnp.int32),
            pltpu.VMEM((k, d), jnp.float32),
            pltpu.VMEM((k, d), jnp.float32),
            pltpu.VMEM((80, d), jnp.float32),
            pltpu.VMEM_SHARED((n, d), jnp.float32),
        ],
    )


def kernel(x, edge_attr, W_enc, b_enc, W1, b1, g1, be1, W2, b2, g2, be2,
           Wp, bp, edge_index, batch):
    n, d = x.shape
    e = edge_attr.shape[0]
    num_layers = W1.shape[0]
    c_out = Wp.shape[1]

    src = edge_index[0]
    dst = edge_index[1]

    h = pl.pallas_call(
        _enc_body,
        out_shape=jax.ShapeDtypeStruct((n, d), jnp.float32),
    )(x, W_enc, b_enc.reshape(1, d))

    edge_fn = _make_edge_kernel(n, d, e)

    for l in range(num_layers):
        parts = edge_fn(h, edge_attr, src, dst)
        h = pl.pallas_call(
            functools.partial(_mlp_body, l < num_layers - 1),
            out_shape=jax.ShapeDtypeStruct((n, d), jnp.float32),
        )(h, parts, W1[l], b1[l].reshape(1, -1), g1[l].reshape(1, -1),
          be1[l].reshape(1, -1), W2[l], b2[l].reshape(1, -1),
          g2[l].reshape(1, -1), be2[l].reshape(1, -1))

    out = pl.pallas_call(
        _pool_body,
        out_shape=jax.ShapeDtypeStruct((_G, c_out), jnp.float32),
    )(h, batch.reshape(1, n), Wp, bp.reshape(1, c_out))
    return out


# speedup vs baseline: 7.5905x; 2.4786x over previous
import functools

import jax
import jax.numpy as jnp
from jax import lax
from jax.experimental import pallas as pl
from jax.experimental.pallas import tpu as pltpu
from jax.experimental.pallas import tpu_sc as plsc

_EPS = 1e-5
_G = 64


def _enc_body(x_ref, w_ref, b_ref, o_ref):
    o_ref[...] = (
        jnp.dot(x_ref[...], w_ref[...], preferred_element_type=jnp.float32)
        + b_ref[...]
    )


def _mlp_body(relu_out, h_ref, p_ref, w1_ref, b1_ref, g1_ref, be1_ref,
              w2_ref, b2_ref, g2_ref, be2_ref, o_ref):
    p = p_ref[...]
    z = h_ref[...] + p[0] + p[1]
    u = jnp.dot(z, w1_ref[...], preferred_element_type=jnp.float32) + b1_ref[...]
    m1 = jnp.mean(u, axis=0, keepdims=True)
    v1 = jnp.mean((u - m1) ** 2, axis=0, keepdims=True)
    u = (u - m1) / jnp.sqrt(v1 + _EPS) * g1_ref[...] + be1_ref[...]
    u = jnp.maximum(u, 0.0)
    y = jnp.dot(u, w2_ref[...], preferred_element_type=jnp.float32) + b2_ref[...]
    m2 = jnp.mean(y, axis=0, keepdims=True)
    v2 = jnp.mean((y - m2) ** 2, axis=0, keepdims=True)
    y = (y - m2) / jnp.sqrt(v2 + _EPS) * g2_ref[...] + be2_ref[...]
    if relu_out:
        y = jnp.maximum(y, 0.0)
    o_ref[...] = y


def _pool_body(h_ref, b_ref, wp_ref, bp_ref, o_ref):
    h = h_ref[...]
    batch_row = b_ref[...]
    g = o_ref.shape[0]
    n = h.shape[0]
    onehot_t = (lax.broadcasted_iota(jnp.int32, (g, n), 0) == batch_row
                ).astype(jnp.float32)
    sums = jnp.dot(onehot_t, h, preferred_element_type=jnp.float32)
    counts = jnp.sum(onehot_t, axis=1, keepdims=True)
    hg = sums / jnp.maximum(counts, 1.0)
    o_ref[...] = (
        jnp.dot(hg, wp_ref[...], preferred_element_type=jnp.float32)
        + bp_ref[...]
    )


_EK_K = 40


@functools.lru_cache(maxsize=None)
def _make_edge_kernel(n, d, e):
    nc, ns = 2, 16
    nw = nc * ns
    epw = e // nw
    k = _EK_K
    nblk = epw // k
    zr = 40
    nchunk = n // zr
    chunk_iters = (nchunk + ns - 1) // ns
    assert epw % k == 0 and n % zr == 0 and nblk % 2 == 0 and nblk >= 6

    mesh = plsc.VectorSubcoreMesh(core_axis_name="c", subcore_axis_name="s",
                                  num_cores=nc, num_subcores=ns)

    def body(h_hbm, ea_hbm, src_hbm, dst_hbm, out_hbm,
             si0, si1, si2, si3, di0, di1, di2, di3,
             gth0, gth1, ea0, ea1, msg0, msg1, zbuf, acc,
             isem0, isem1, isem2, isem3,
             gsem0, gsem1, esem0, esem1, ssem0, ssem1, zsem):
        c = lax.axis_index("c")
        s = lax.axis_index("s")
        w = c * ns + s
        base = w * epw

        sis = (si0, si1, si2, si3)
        dis = (di0, di1, di2, di3)
        isems = (isem0, isem1, isem2, isem3)
        gths = (gth0, gth1)
        eas = (ea0, ea1)
        msgs = (msg0, msg1)
        gsems = (gsem0, gsem1)
        esems = (esem0, esem1)
        ssems = (ssem0, ssem1)

        @pl.loop(0, zr)
        def _(i):
            for j in range(0, d, 16):
                zbuf[i, pl.ds(j, 16)] = jnp.zeros((16,), jnp.float32)

        @pl.loop(0, chunk_iters)
        def _(r):
            m = s + r * ns

            @pl.when(m < nchunk)
            def _():
                pltpu.async_copy(zbuf, acc.at[pl.ds(m * zr, zr)], zsem)

        @pl.loop(0, chunk_iters)
        def _(r):
            m = s + r * ns

            @pl.when(m < nchunk)
            def _():
                pltpu.make_async_copy(zbuf, acc.at[pl.ds(m * zr, zr)],
                                      zsem).wait()

        plsc.subcore_barrier()

        def issue_idx(bb, q):
            eoff = pl.multiple_of(base + bb * k, 8)
            pltpu.async_copy(src_hbm.at[pl.ds(eoff, k)], sis[q], isems[q])
            pltpu.async_copy(dst_hbm.at[pl.ds(eoff, k)], dis[q], isems[q])

        def wait_idx(bb, q):
            eoff = pl.multiple_of(base + bb * k, 8)
            pltpu.make_async_copy(src_hbm.at[pl.ds(eoff, k)], sis[q],
                                  isems[q]).wait()
            pltpu.make_async_copy(dst_hbm.at[pl.ds(eoff, k)], dis[q],
                                  isems[q]).wait()

        def issue_data(bb, b, q):
            pltpu.async_copy(h_hbm.at[sis[q]], gths[b], gsems[b])
            eoff = pl.multiple_of(base + bb * k, 8)
            pltpu.async_copy(ea_hbm.at[pl.ds(eoff, k)], eas[b], esems[b])

        def wait_data(bb, b, q):
            pltpu.make_async_copy(h_hbm.at[sis[q]], gths[b], gsems[b]).wait()
            eoff = pl.multiple_of(base + bb * k, 8)
            pltpu.make_async_copy(ea_hbm.at[pl.ds(eoff, k)], eas[b],
                                  esems[b]).wait()

        def compute(b):
            gth, ea, msg = gths[b], eas[b], msgs[b]

            @pl.loop(0, k)
            def _(i):
                for j in range(0, d, 16):
                    v = gth[i, pl.ds(j, 16)] + ea[i, pl.ds(j, 16)]
                    msg[i, pl.ds(j, 16)] = jnp.maximum(v, 0.0)

        def issue_scatter(b, q):
            pltpu.async_copy(msgs[b], acc.at[dis[q]], ssems[b], add=True)

        def wait_scatter(b, q):
            pltpu.make_async_copy(msgs[b], acc.at[dis[q]], ssems[b]).wait()

        for bb in range(4):
            issue_idx(bb, bb)
        for bb in range(2):
            wait_idx(bb, bb)
            issue_data(bb, bb, bb)

        for bb in range(2):
            b = bb
            wait_data(bb, b, bb)
            compute(b)
            issue_scatter(b, bb)
            wait_idx(bb + 2, bb + 2)
            issue_data(bb + 2, b, bb + 2)

        @pl.loop(0, (nblk - 2) // 4)
        def _(t):
            for p in range(4):
                b = p % 2
                q = (2 + p) % 4
                bb = 4 * t + 2 + p
                wait_scatter(b, p)
                nxt = bb + 2

                @pl.when(nxt < nblk)
                def _():
                    issue_idx(nxt, p)

                wait_data(bb, b, q)
                compute(b)
                issue_scatter(b, q)

                @pl.when(nxt < nblk)
                def _():
                    wait_idx(nxt, p)
                    issue_data(nxt, b, p)

        wait_scatter(0, (nblk - 2) % 4)
        wait_scatter(1, (nblk - 1) % 4)

        plsc.subcore_barrier()

        @pl.loop(0, chunk_iters)
        def _(r):
            m = s + r * ns

            @pl.when(m < nchunk)
            def _():
                rr = pl.multiple_of(m * zr, 8)
                pltpu.async_copy(acc.at[pl.ds(rr, zr)],
                                 out_hbm.at[c, pl.ds(rr, zr)], zsem)

        @pl.loop(0, chunk_iters)
        def _(r):
            m = s + r * ns

            @pl.when(m < nchunk)
            def _():
                rr = pl.multiple_of(m * zr, 8)
                pltpu.make_async_copy(acc.at[pl.ds(rr, zr)],
                                      out_hbm.at[c, pl.ds(rr, zr)],
                                      zsem).wait()

    return pl.kernel(
        body,
        out_type=jax.ShapeDtypeStruct((nc, n, d), jnp.float32),
        mesh=mesh,
        scratch_types=[
            pltpu.VMEM((k,), jnp.int32), pltpu.VMEM((k,), jnp.int32),
            pltpu.VMEM((k,), jnp.int32), pltpu.VMEM((k,), jnp.int32),
            pltpu.VMEM((k,), jnp.int32), pltpu.VMEM((k,), jnp.int32),
            pltpu.VMEM((k,), jnp.int32), pltpu.VMEM((k,), jnp.int32),
            pltpu.VMEM((k, d), jnp.float32), pltpu.VMEM((k, d), jnp.float32),
            pltpu.VMEM((k, d), jnp.float32), pltpu.VMEM((k, d), jnp.float32),
            pltpu.VMEM((k, d), jnp.float32), pltpu.VMEM((k, d), jnp.float32),
            pltpu.VMEM((zr, d), jnp.float32),
            pltpu.VMEM_SHARED((n, d), jnp.float32),
            pltpu.SemaphoreType.DMA, pltpu.SemaphoreType.DMA,
            pltpu.SemaphoreType.DMA, pltpu.SemaphoreType.DMA,
            pltpu.SemaphoreType.DMA, pltpu.SemaphoreType.DMA,
            pltpu.SemaphoreType.DMA, pltpu.SemaphoreType.DMA,
            pltpu.SemaphoreType.DMA, pltpu.SemaphoreType.DMA,
            pltpu.SemaphoreType.DMA,
        ],
    )


def kernel(x, edge_attr, W_enc, b_enc, W1, b1, g1, be1, W2, b2, g2, be2,
           Wp, bp, edge_index, batch):
    n, d = x.shape
    e = edge_attr.shape[0]
    num_layers = W1.shape[0]
    c_out = Wp.shape[1]

    src = edge_index[0]
    dst = edge_index[1]

    h = pl.pallas_call(
        _enc_body,
        out_shape=jax.ShapeDtypeStruct((n, d), jnp.float32),
    )(x, W_enc, b_enc.reshape(1, d))

    edge_fn = _make_edge_kernel(n, d, e)

    for l in range(num_layers):
        parts = edge_fn(h, edge_attr, src, dst)
        h = pl.pallas_call(
            functools.partial(_mlp_body, l < num_layers - 1),
            out_shape=jax.ShapeDtypeStruct((n, d), jnp.float32),
        )(h, parts, W1[l], b1[l].reshape(1, -1), g1[l].reshape(1, -1),
          be1[l].reshape(1, -1), W2[l], b2[l].reshape(1, -1),
          g2[l].reshape(1, -1), be2[l].reshape(1, -1))

    out = pl.pallas_call(
        _pool_body,
        out_shape=jax.ShapeDtypeStruct((_G, c_out), jnp.float32),
    )(h, batch.reshape(1, n), Wp, bp.reshape(1, c_out))
    return out
